# baseline (speedup 1.0000x reference)
import functools

import jax
import jax.numpy as jnp
from jax import lax
from jax.experimental import pallas as pl
from jax.experimental.pallas import tpu as pltpu
from jax.experimental.pallas import tpu_sc as plsc

_MASK_RATIO = 0.75

_NC = 2
_NS = 16
_NW = _NC * _NS

_C = 48
_L = 16


def _shuffle_constants(B, N):
    len_keep = int(N * (1 - _MASK_RATIO))
    noise = jax.random.uniform(jax.random.key(1), (B, N), dtype=jnp.float32)
    ids_shuffle = jnp.argsort(noise, axis=1)
    ids_restore = jnp.argsort(ids_shuffle, axis=1)
    ids_keep = ids_shuffle[:, :len_keep].astype(jnp.int32)
    mask = ids_restore >= len_keep
    return ids_keep, mask, ids_restore.astype(jnp.int32), len_keep


def _lane_gather(xt_flat, ids_keep, B, N, D, K):
    bpw = B // _NW
    nch = D // _C
    ngr = K // _L
    mesh = plsc.VectorSubcoreMesh(
        core_axis_name="c", subcore_axis_name="s",
        num_cores=_NC, num_subcores=_NS)

    @functools.partial(
        pl.kernel,
        out_type=jax.ShapeDtypeStruct((B * D * K,), jnp.float32),
        mesh=mesh,
        scratch_types=[
            [pltpu.VMEM((_C * N,), jnp.float32)] * 2,
            [pltpu.VMEM((_C * K,), jnp.float32)] * 2,
            pltpu.VMEM((K,), jnp.int32),
            [pltpu.SemaphoreType.DMA] * 2,
            [pltpu.SemaphoreType.DMA] * 2,
        ],
        compiler_params=pltpu.CompilerParams(needs_layout_passes=False),
    )
    def k(x_hbm, idx_hbm, out_hbm, inb, outb, idxv, gsem, ssem):
        wid = lax.axis_index("s") * _NC + lax.axis_index("c")
        base = wid * bpw * D
        n_it = bpw * nch
        oc = [lax.iota(jnp.int32, _L) + (_L * g) for g in range(ngr)]

        def start_in(i):
            bi, c = divmod(i, nch)
            s = i % 2
            return pltpu.async_copy(
                x_hbm.at[pl.ds((base + bi * D + c * _C) * N, _C * N)],
                inb[s], gsem[s])

        in_h = [None] * n_it
        out_h = [None, None]
        cols = None
        in_h[0] = start_in(0)
        for i in range(n_it):
            bi, c = divmod(i, nch)
            s = i % 2
            if c == 0:
                pltpu.sync_copy(idx_hbm.at[wid * bpw + bi], idxv)
                cols = [idxv[pl.ds(_L * g, _L)] for g in range(ngr)]
            if i + 1 < n_it:
                in_h[i + 1] = start_in(i + 1)
            in_h[i].wait()
            if out_h[s] is not None:
                out_h[s].wait()

            def body(r, carry, _cols=cols, _s=s):
                rbase, obase = carry
                for g in range(ngr):
                    v = plsc.load_gather(inb[_s], [rbase + _cols[g]])
                    plsc.store_scatter(outb[_s], [obase + oc[g]], v)
                return (rbase + N, obase + K)

            zero = jnp.zeros((_L,), dtype=jnp.int32)
            lax.fori_loop(0, _C, body, (zero, zero))
            out_h[s] = pltpu.async_copy(
                outb[s],
                out_hbm.at[pl.ds((base + bi * D + c * _C) * K, _C * K)],
                ssem[s])
        out_h[0].wait()
        out_h[1].wait()

    return k(xt_flat, ids_keep)


def kernel(x):
    B, N, D = x.shape
    ids_keep, mask, ids_restore, len_keep = _shuffle_constants(B, N)
    xt_flat = x.transpose(0, 2, 1).reshape(B * D * N)
    out_t = _lane_gather(xt_flat, ids_keep, B, N, D, len_keep)
    x_masked = out_t.reshape(B, D, len_keep).transpose(0, 2, 1)
    return (x_masked, mask, ids_restore)


# speedup vs baseline: 1.7082x; 1.7082x over previous
import functools

import jax
import jax.numpy as jnp
from jax import lax
from jax.experimental import pallas as pl
from jax.experimental.pallas import tpu as pltpu
from jax.experimental.pallas import tpu_sc as plsc

_MASK_RATIO = 0.75

_NC = 2
_NS = 16
_NW = _NC * _NS

_C = 48
_L = 16


def _shuffle_constants(B, N):
    len_keep = int(N * (1 - _MASK_RATIO))
    noise = jax.random.uniform(jax.random.key(1), (B, N), dtype=jnp.float32)
    ids_shuffle = jnp.argsort(noise, axis=1)
    ids_restore = jnp.argsort(ids_shuffle, axis=1)
    ids_keep = ids_shuffle[:, :len_keep].astype(jnp.int32)
    mask = ids_restore >= len_keep
    return ids_keep, mask, ids_restore.astype(jnp.int32), len_keep


def _lane_gather(xt_flat, ids_keep, B, N, D, K):
    bpw = B // _NW
    nch = D // _C
    ngr = K // _L
    mesh = plsc.VectorSubcoreMesh(
        core_axis_name="c", subcore_axis_name="s",
        num_cores=_NC, num_subcores=_NS)

    @functools.partial(
        pl.kernel,
        out_type=jax.ShapeDtypeStruct((B * D, K), jnp.float32),
        mesh=mesh,
        scratch_types=[
            pltpu.VMEM((2, _C, N), jnp.float32),
            pltpu.VMEM((2, _C, K), jnp.float32),
            pltpu.VMEM((K,), jnp.int32),
            [pltpu.SemaphoreType.DMA] * 2,
            [pltpu.SemaphoreType.DMA] * 2,
        ],
        compiler_params=pltpu.CompilerParams(needs_layout_passes=False),
    )
    def k(x_hbm, idx_hbm, out_hbm, inb, outb, idxv, gsem, ssem):
        wid = lax.axis_index("s") * _NC + lax.axis_index("c")
        base = wid * bpw * D
        n_it = bpw * nch

        def start_in(i):
            bi, c = divmod(i, nch)
            s = i % 2
            return pltpu.async_copy(
                x_hbm.at[pl.ds(base + bi * D + c * _C, _C)],
                inb.at[s], gsem[s])

        in_h = [None] * n_it
        out_h = [None, None]
        cols = None
        in_h[0] = start_in(0)
        for i in range(n_it):
            bi, c = divmod(i, nch)
            s = i % 2
            if c == 0:
                pltpu.sync_copy(idx_hbm.at[wid * bpw + bi], idxv)
                cols = [idxv[pl.ds(_L * g, _L)] for g in range(ngr)]
            if i + 1 < n_it:
                in_h[i + 1] = start_in(i + 1)
            in_h[i].wait()
            if out_h[s] is not None:
                out_h[s].wait()

            def body(r, carry, _cols=cols, _s=s):
                rr = jnp.full((_L,), r, dtype=jnp.int32)
                for g in range(ngr):
                    v = plsc.load_gather(inb.at[_s], [rr, _cols[g]])
                    oc = lax.iota(jnp.int32, _L) + (_L * g)
                    plsc.store_scatter(outb.at[_s], [rr, oc], v)
                return carry

            lax.fori_loop(0, _C, body, 0)
            out_h[s] = pltpu.async_copy(
                outb.at[s],
                out_hbm.at[pl.ds(base + bi * D + c * _C, _C)], ssem[s])
        out_h[0].wait()
        out_h[1].wait()

    return k(xt_flat, ids_keep)


def kernel(x):
    B, N, D = x.shape
    ids_keep, mask, ids_restore, len_keep = _shuffle_constants(B, N)
    xt_flat = x.transpose(0, 2, 1).reshape(B * D, N)
    out_t = _lane_gather(xt_flat, ids_keep, B, N, D, len_keep)
    x_masked = out_t.reshape(B, D, len_keep).transpose(0, 2, 1)
    return (x_masked, mask, ids_restore)
